# baseline (speedup 1.0000x reference)
import functools

import jax
import jax.numpy as jnp
from jax import lax
from jax.experimental import pallas as pl
from jax.experimental.pallas import tpu as pltpu
from jax.experimental.pallas import tpu_sc as plsc

N = 10000
E = 320000
D = 128
H = 128
OUT = 64

NC = 2
NS = 16
NW = NC * NS

NPAD = 10240
C = 128
EPAD = 323584
CHUNKS_PER_TILE = EPAD // (NW * C)
TOTAL_CHUNKS = EPAD // C
K0 = 148
K1 = TOTAL_CHUNKS // NS - K0
KMAX = max(K0, K1)
EDGES_PER_TILE = EPAD // NW
ROWS_PER_SC_TILE = NPAD // NS

RB = 1280
GRID = NPAD // RB

_mesh = plsc.VectorSubcoreMesh(core_axis_name="c", subcore_axis_name="s")


@functools.partial(
    pl.kernel,
    out_type=jax.ShapeDtypeStruct((NW, 2, NPAD), jnp.float32),
    mesh=_mesh,
    scratch_types=[
        pltpu.VMEM((EDGES_PER_TILE,), jnp.int32),
        pltpu.VMEM((EDGES_PER_TILE,), jnp.int32),
        pltpu.VMEM((NPAD,), jnp.float32),
        pltpu.VMEM((NPAD,), jnp.float32),
    ],
    compiler_params=pltpu.CompilerParams(needs_layout_passes=False),
)
def _sc_degrees(src_hbm, dst_hbm, out_hbm, sidx, didx, ldo, ldi):
    cid = lax.axis_index("c")
    sid = lax.axis_index("s")
    wid = sid * NC + cid

    zero16 = jnp.zeros((16,), jnp.float32)
    ones16 = jnp.ones((16,), jnp.float32)

    def zbody(i, _):
        ldo[pl.ds(i * 16, 16)] = zero16
        ldi[pl.ds(i * 16, 16)] = zero16
        return 0

    lax.fori_loop(0, NPAD // 16, zbody, 0)

    e0 = wid * EDGES_PER_TILE
    pltpu.sync_copy(src_hbm.at[pl.ds(e0, EDGES_PER_TILE)], sidx)
    pltpu.sync_copy(dst_hbm.at[pl.ds(e0, EDGES_PER_TILE)], didx)

    def body(i, _):
        sv = sidx[pl.ds(i * 16, 16)]
        dv = didx[pl.ds(i * 16, 16)]
        plsc.addupdate_scatter(ldo, [sv], ones16)
        plsc.addupdate_scatter(ldi, [dv], ones16)
        return 0

    lax.fori_loop(0, EDGES_PER_TILE // 16, body, 0)

    pltpu.sync_copy(ldo, out_hbm.at[wid, 0])
    pltpu.sync_copy(ldi, out_hbm.at[wid, 1])


NIDX = 3
NROW = 2


@functools.partial(
    pl.kernel,
    out_type=jax.ShapeDtypeStruct((NC, NPAD, D), jnp.float32),
    mesh=_mesh,
    scratch_types=(
        [pltpu.VMEM((1, C), jnp.int32) for _ in range(NIDX)]
        + [pltpu.VMEM((1, C), jnp.int32) for _ in range(NIDX)]
        + [pltpu.VMEM((C, D), jnp.float32) for _ in range(NROW)]
        + [pltpu.VMEM_SHARED((NPAD, D), jnp.float32)]
        + [pltpu.SemaphoreType.DMA for _ in range(NIDX + 2 * NROW)]
    ),
    compiler_params=pltpu.CompilerParams(needs_layout_passes=False),
)
def _sc_agg(xs_hbm, src4d_hbm, dst4d_hbm, zeros_hbm, out_hbm,
            si0, si1, si2, di0, di1, di2, rows0, rows1, acc,
            isem0, isem1, isem2, gsem0, gsem1, ssem0, ssem1):
    sbuf = [si0, si1, si2]
    dbuf = [di0, di1, di2]
    rows = [rows0, rows1]
    isem = [isem0, isem1, isem2]
    gsem = [gsem0, gsem1]
    ssem = [ssem0, ssem1]
    cid = lax.axis_index("c")
    sid = lax.axis_index("s")

    nck = sid * K0 + cid * (NS * K0 + sid * (K1 - K0))
    kc = K0 + cid * (K1 - K0)

    def idx_copy(c, sl):
        pltpu.async_copy(src4d_hbm.at[nck + c], sbuf[sl], isem[sl])
        pltpu.async_copy(dst4d_hbm.at[nck + c], dbuf[sl], isem[sl])

    def idx_wait(c, sl):
        pltpu.make_async_copy(src4d_hbm.at[nck + c], sbuf[sl],
                              isem[sl]).wait()
        pltpu.make_async_copy(dst4d_hbm.at[nck + c], dbuf[sl],
                              isem[sl]).wait()

    def gather(sl, b):
        pltpu.async_copy(xs_hbm.at[sbuf[sl].at[0]], rows[b], gsem[b])

    def gather_wait(sl, b):
        pltpu.make_async_copy(xs_hbm.at[sbuf[sl].at[0]], rows[b],
                              gsem[b]).wait()

    def scatter(sl, b):
        pltpu.async_copy(rows[b], acc.at[dbuf[sl].at[0]], ssem[b], add=True)

    def scatter_wait(sl, b):
        pltpu.make_async_copy(rows[b], acc.at[dbuf[sl].at[0]],
                              ssem[b]).wait()

    idx_copy(0, 0)
    idx_copy(1, 1)
    r0 = sid * ROWS_PER_SC_TILE
    pltpu.sync_copy(zeros_hbm, acc.at[pl.ds(r0, ROWS_PER_SC_TILE)])
    idx_wait(0, 0)
    gather(0, 0)
    plsc.subcore_barrier()

    UNROLL = 6

    def body(g, _):
        for u in range(UNROLL):
            c = g * UNROLL + u

            @pl.when(c + 2 < kc)
            def _():
                idx_copy(c + 2, (u + 2) % NIDX)

            @pl.when((c + 1 < kc) & (c >= 1))
            def _():
                scatter_wait((u + 1) % NIDX, (u + 1) % NROW)

            @pl.when(c + 1 < kc)
            def _():
                idx_wait(c + 1, (u + 1) % NIDX)
                gather((u + 1) % NIDX, (u + 1) % NROW)

            @pl.when(c < kc)
            def _():
                gather_wait(u % NIDX, u % NROW)
                scatter(u % NIDX, u % NROW)
        return 0

    lax.fori_loop(0, (KMAX + UNROLL - 1) // UNROLL, body, 0)

    scatter_wait(0, 0)
    scatter_wait(1, 1)

    plsc.subcore_barrier()
    pltpu.sync_copy(acc.at[pl.ds(r0, ROWS_PER_SC_TILE)],
                    out_hbm.at[cid, pl.ds(r0, ROWS_PER_SC_TILE)])


def _tc_prep_body(degp_ref, x_ref, xs_ref, norms_ref):
    deg = jnp.sum(degp_ref[...], axis=0)
    norms = lax.rsqrt(jnp.maximum(deg, 1.0))
    norms_ref[...] = norms
    xs_ref[...] = x_ref[...] * norms[0][:, None]


def _tc_prep(deg_partials, x_pad):
    return pl.pallas_call(
        _tc_prep_body,
        grid=(GRID,),
        in_specs=[
            pl.BlockSpec((NW, 2, RB), lambda i: (0, 0, i)),
            pl.BlockSpec((RB, D), lambda i: (i, 0)),
        ],
        out_specs=[
            pl.BlockSpec((RB, D), lambda i: (i, 0)),
            pl.BlockSpec((2, RB), lambda i: (0, i)),
        ],
        out_shape=[
            jax.ShapeDtypeStruct((NPAD, D), jnp.float32),
            jax.ShapeDtypeStruct((2, NPAD), jnp.float32),
        ],
        compiler_params=pltpu.CompilerParams(
            dimension_semantics=("arbitrary",)),
    )(deg_partials, x_pad)


def _tc_layer_body(agg_ref, norms_ref, W_ref, b_ref, out_ref):
    i = pl.program_id(0)
    a = (agg_ref[0] + agg_ref[1]) * norms_ref[1][:, None]
    h = jnp.dot(a, W_ref[...], preferred_element_type=jnp.float32,
                precision=lax.Precision.HIGHEST) + b_ref[...]
    h = jnp.maximum(h, 0.0) * norms_ref[0][:, None]
    ridx = i * RB + lax.broadcasted_iota(jnp.int32, (RB, 1), 0)
    out_ref[...] = jnp.where(ridx < N, h, 0.0)


def _tc_layer(agg_partials, norms, W1, b1_2d):
    return pl.pallas_call(
        _tc_layer_body,
        grid=(GRID,),
        in_specs=[
            pl.BlockSpec((2, RB, D), lambda i: (0, i, 0)),
            pl.BlockSpec((2, RB), lambda i: (0, i)),
            pl.BlockSpec((D, H), lambda i: (0, 0)),
            pl.BlockSpec((1, H), lambda i: (0, 0)),
        ],
        out_specs=pl.BlockSpec((RB, H), lambda i: (i, 0)),
        out_shape=jax.ShapeDtypeStruct((NPAD, H), jnp.float32),
        compiler_params=pltpu.CompilerParams(
            dimension_semantics=("arbitrary",)),
    )(agg_partials, norms, W1, b1_2d)


def _tc_final_body(agg_ref, norms_ref, W_ref, b_ref, Wfc_ref, bfc_ref,
                   out_ref, acc_ref):
    i = pl.program_id(0)
    a = (agg_ref[0] + agg_ref[1]) * norms_ref[1][:, None]
    h = jnp.dot(a, W_ref[...], preferred_element_type=jnp.float32,
                precision=lax.Precision.HIGHEST) + b_ref[...]
    h = jnp.maximum(h, 0.0)
    ridx = i * RB + lax.broadcasted_iota(jnp.int32, (RB, 1), 0)
    h = jnp.where(ridx < N, h, 0.0)
    s = jnp.sum(h, axis=0, keepdims=True)

    @pl.when(i == 0)
    def _():
        acc_ref[...] = jnp.zeros_like(acc_ref)

    acc_ref[...] += s

    @pl.when(i == GRID - 1)
    def _():
        hg = acc_ref[...] * (1.0 / N)
        out_ref[...] = jnp.dot(hg, Wfc_ref[...],
                               preferred_element_type=jnp.float32,
                               precision=lax.Precision.HIGHEST) + bfc_ref[...]


def _tc_final(agg_partials, norms, W1, b1_2d, Wfc, bfc_2d):
    return pl.pallas_call(
        _tc_final_body,
        grid=(GRID,),
        in_specs=[
            pl.BlockSpec((2, RB, D), lambda i: (0, i, 0)),
            pl.BlockSpec((2, RB), lambda i: (0, i)),
            pl.BlockSpec((D, H), lambda i: (0, 0)),
            pl.BlockSpec((1, H), lambda i: (0, 0)),
            pl.BlockSpec((H, OUT), lambda i: (0, 0)),
            pl.BlockSpec((1, OUT), lambda i: (0, 0)),
        ],
        out_specs=pl.BlockSpec((1, OUT), lambda i: (0, 0)),
        out_shape=jax.ShapeDtypeStruct((1, OUT), jnp.float32),
        scratch_shapes=[pltpu.VMEM((1, H), jnp.float32)],
        compiler_params=pltpu.CompilerParams(
            dimension_semantics=("arbitrary",)),
    )(agg_partials, norms, W1, b1_2d, Wfc, bfc_2d)


def kernel(x, edge_index, W1, b1, Wfc, bfc):
    src = edge_index[0]
    dst = edge_index[1]
    pad = jnp.full((EPAD - E,), N, dtype=jnp.int32)
    src_p = jnp.concatenate([src, pad])
    dst_p = jnp.concatenate([dst, pad])
    src_2d = src_p.reshape(TOTAL_CHUNKS, 1, C)
    dst_2d = dst_p.reshape(TOTAL_CHUNKS, 1, C)
    x_pad = jnp.zeros((NPAD, D), jnp.float32).at[:N].set(x)
    zeros_tile = jnp.zeros((ROWS_PER_SC_TILE, D), jnp.float32)
    b1_2d = b1.reshape(1, H)
    bfc_2d = bfc.reshape(1, OUT)

    degp = _sc_degrees(src_p, dst_p)
    xs, norms = _tc_prep(degp, x_pad)
    agg1 = _sc_agg(xs, src_2d, dst_2d, zeros_tile)
    h1s = _tc_layer(agg1, norms, W1, b1_2d)
    agg2 = _sc_agg(h1s, src_2d, dst_2d, zeros_tile)
    return _tc_final(agg2, norms, W1, b1_2d, Wfc, bfc_2d)


# speedup vs baseline: 1.2803x; 1.2803x over previous
import functools

import jax
import jax.numpy as jnp
from jax import lax
from jax.experimental import pallas as pl
from jax.experimental.pallas import tpu as pltpu
from jax.experimental.pallas import tpu_sc as plsc

N = 10000
E = 320000
D = 128
H = 128
OUT = 64

NC = 2
NS = 16
NW = NC * NS

NPAD = 10240
C = 112
EPAD = 322560
TOTAL_CHUNKS = EPAD // C
K0 = 168
K1 = TOTAL_CHUNKS // NS - K0
KMAX = max(K0, K1)
EDGES_PER_TILE_DEG = E // NW
ROWS_PER_SC_TILE = NPAD // NS

RB = 1280
GRID = NPAD // RB

_mesh = plsc.VectorSubcoreMesh(core_axis_name="c", subcore_axis_name="s")


@functools.partial(
    pl.kernel,
    out_type=jax.ShapeDtypeStruct((NW, 2, NPAD), jnp.float32),
    mesh=_mesh,
    scratch_types=[
        pltpu.VMEM((EDGES_PER_TILE_DEG,), jnp.int32),
        pltpu.VMEM((EDGES_PER_TILE_DEG,), jnp.int32),
        pltpu.VMEM((NPAD,), jnp.float32),
        pltpu.VMEM((NPAD,), jnp.float32),
    ],
    compiler_params=pltpu.CompilerParams(needs_layout_passes=False),
)
def _sc_degrees(src_hbm, dst_hbm, out_hbm, sidx, didx, ldo, ldi):
    cid = lax.axis_index("c")
    sid = lax.axis_index("s")
    wid = sid * NC + cid

    zero16 = jnp.zeros((16,), jnp.float32)
    ones16 = jnp.ones((16,), jnp.float32)

    def zbody(i, _):
        ldo[pl.ds(i * 16, 16)] = zero16
        ldi[pl.ds(i * 16, 16)] = zero16
        return 0

    lax.fori_loop(0, NPAD // 16, zbody, 0)

    e0 = wid * EDGES_PER_TILE_DEG
    pltpu.sync_copy(src_hbm.at[pl.ds(e0, EDGES_PER_TILE_DEG)], sidx)
    pltpu.sync_copy(dst_hbm.at[pl.ds(e0, EDGES_PER_TILE_DEG)], didx)

    def body(i, _):
        sv = sidx[pl.ds(i * 16, 16)]
        dv = didx[pl.ds(i * 16, 16)]
        plsc.addupdate_scatter(ldo, [sv], ones16)
        plsc.addupdate_scatter(ldi, [dv], ones16)
        return 0

    lax.fori_loop(0, EDGES_PER_TILE_DEG // 16, body, 0)

    pltpu.sync_copy(ldo, out_hbm.at[wid, 0])
    pltpu.sync_copy(ldi, out_hbm.at[wid, 1])


NIDX = 4
NROW = 3


@functools.partial(
    pl.kernel,
    out_type=jax.ShapeDtypeStruct((NC, NPAD, D), jnp.float32),
    mesh=_mesh,
    scratch_types=(
        [pltpu.VMEM((2, C), jnp.int32) for _ in range(NIDX)]
        + [pltpu.VMEM((C, D), jnp.float32) for _ in range(NROW)]
        + [pltpu.VMEM_SHARED((NPAD, D), jnp.float32)]
        + [pltpu.SemaphoreType.DMA for _ in range(NIDX + 2 * NROW)]
    ),
    compiler_params=pltpu.CompilerParams(needs_layout_passes=False),
)
def _sc_agg(xs_hbm, edg_hbm, zeros_hbm, out_hbm,
            ib0, ib1, ib2, ib3, rows0, rows1, rows2, acc,
            isem0, isem1, isem2, isem3, gsem0, gsem1, gsem2,
            ssem0, ssem1, ssem2):
    ibuf = [ib0, ib1, ib2, ib3]
    rows = [rows0, rows1, rows2]
    isem = [isem0, isem1, isem2, isem3]
    gsem = [gsem0, gsem1, gsem2]
    ssem = [ssem0, ssem1, ssem2]
    cid = lax.axis_index("c")
    sid = lax.axis_index("s")

    nck = sid * K0 + cid * (NS * K0 + sid * (K1 - K0))
    kc = K0 + cid * (K1 - K0)

    def idx_copy(c, sl):
        pltpu.async_copy(edg_hbm.at[nck + c], ibuf[sl], isem[sl])

    def idx_wait(c, sl):
        pltpu.make_async_copy(edg_hbm.at[nck + c], ibuf[sl], isem[sl]).wait()

    def gather(sl, b):
        pltpu.async_copy(xs_hbm.at[ibuf[sl].at[0]], rows[b], gsem[b])

    def gather_wait(sl, b):
        pltpu.make_async_copy(xs_hbm.at[ibuf[sl].at[0]], rows[b],
                              gsem[b]).wait()

    def scatter(sl, b):
        pltpu.async_copy(rows[b], acc.at[ibuf[sl].at[1]], ssem[b], add=True)

    def scatter_wait(sl, b):
        pltpu.make_async_copy(rows[b], acc.at[ibuf[sl].at[1]],
                              ssem[b]).wait()

    idx_copy(0, 0)
    idx_copy(1, 1)
    idx_copy(2, 2)
    r0 = sid * ROWS_PER_SC_TILE
    pltpu.sync_copy(zeros_hbm, acc.at[pl.ds(r0, ROWS_PER_SC_TILE)])
    idx_wait(0, 0)
    gather(0, 0)
    idx_wait(1, 1)
    gather(1, 1)
    plsc.subcore_barrier()

    UNROLL = 12

    def body(g, _):
        for u in range(UNROLL):
            c = g * UNROLL + u

            @pl.when((c + 2 < kc) & (c >= 1))
            def _():
                scatter_wait((u + 3) % NIDX, (u + 2) % NROW)

            @pl.when(c + 3 < kc)
            def _():
                idx_copy(c + 3, (u + 3) % NIDX)

            @pl.when(c + 2 < kc)
            def _():
                idx_wait(c + 2, (u + 2) % NIDX)
                gather((u + 2) % NIDX, (u + 2) % NROW)

            @pl.when(c < kc)
            def _():
                gather_wait(u % NIDX, u % NROW)
                scatter(u % NIDX, u % NROW)
        return 0

    lax.fori_loop(0, (KMAX + UNROLL - 1) // UNROLL, body, 0)

    scatter_wait(0, 0)
    scatter_wait(1, 1)
    scatter_wait(2, 2)

    plsc.subcore_barrier()
    pltpu.sync_copy(acc.at[pl.ds(r0, ROWS_PER_SC_TILE)],
                    out_hbm.at[cid, pl.ds(r0, ROWS_PER_SC_TILE)])


def _tc_prep_body(degp_ref, x_ref, xs_ref, norms_ref):
    deg = jnp.sum(degp_ref[...], axis=0)
    norms = lax.rsqrt(jnp.maximum(deg, 1.0))
    norms_ref[...] = norms
    xs_ref[...] = x_ref[...] * norms[0][:, None]


def _tc_prep(deg_partials, x_pad):
    return pl.pallas_call(
        _tc_prep_body,
        grid=(GRID,),
        in_specs=[
            pl.BlockSpec((NW, 2, RB), lambda i: (0, 0, i)),
            pl.BlockSpec((RB, D), lambda i: (i, 0)),
        ],
        out_specs=[
            pl.BlockSpec((RB, D), lambda i: (i, 0)),
            pl.BlockSpec((2, RB), lambda i: (0, i)),
        ],
        out_shape=[
            jax.ShapeDtypeStruct((NPAD, D), jnp.float32),
            jax.ShapeDtypeStruct((2, NPAD), jnp.float32),
        ],
        compiler_params=pltpu.CompilerParams(
            dimension_semantics=("arbitrary",)),
    )(deg_partials, x_pad)


def _tc_layer_body(agg_ref, norms_ref, W_ref, b_ref, out_ref):
    i = pl.program_id(0)
    a = (agg_ref[0] + agg_ref[1]) * norms_ref[1][:, None]
    h = jnp.dot(a, W_ref[...], preferred_element_type=jnp.float32,
                precision=lax.Precision.HIGHEST) + b_ref[...]
    h = jnp.maximum(h, 0.0) * norms_ref[0][:, None]
    ridx = i * RB + lax.broadcasted_iota(jnp.int32, (RB, 1), 0)
    out_ref[...] = jnp.where(ridx < N, h, 0.0)


def _tc_layer(agg_partials, norms, W1, b1_2d):
    return pl.pallas_call(
        _tc_layer_body,
        grid=(GRID,),
        in_specs=[
            pl.BlockSpec((2, RB, D), lambda i: (0, i, 0)),
            pl.BlockSpec((2, RB), lambda i: (0, i)),
            pl.BlockSpec((D, H), lambda i: (0, 0)),
            pl.BlockSpec((1, H), lambda i: (0, 0)),
        ],
        out_specs=pl.BlockSpec((RB, H), lambda i: (i, 0)),
        out_shape=jax.ShapeDtypeStruct((NPAD, H), jnp.float32),
        compiler_params=pltpu.CompilerParams(
            dimension_semantics=("arbitrary",)),
    )(agg_partials, norms, W1, b1_2d)


def _tc_final_body(agg_ref, norms_ref, W_ref, b_ref, Wfc_ref, bfc_ref,
                   out_ref, acc_ref):
    i = pl.program_id(0)
    a = (agg_ref[0] + agg_ref[1]) * norms_ref[1][:, None]
    h = jnp.dot(a, W_ref[...], preferred_element_type=jnp.float32,
                precision=lax.Precision.HIGHEST) + b_ref[...]
    h = jnp.maximum(h, 0.0)
    ridx = i * RB + lax.broadcasted_iota(jnp.int32, (RB, 1), 0)
    h = jnp.where(ridx < N, h, 0.0)
    s = jnp.sum(h, axis=0, keepdims=True)

    @pl.when(i == 0)
    def _():
        acc_ref[...] = jnp.zeros_like(acc_ref)

    acc_ref[...] += s

    @pl.when(i == GRID - 1)
    def _():
        hg = acc_ref[...] * (1.0 / N)
        out_ref[...] = jnp.dot(hg, Wfc_ref[...],
                               preferred_element_type=jnp.float32,
                               precision=lax.Precision.HIGHEST) + bfc_ref[...]


def _tc_final(agg_partials, norms, W1, b1_2d, Wfc, bfc_2d):
    return pl.pallas_call(
        _tc_final_body,
        grid=(GRID,),
        in_specs=[
            pl.BlockSpec((2, RB, D), lambda i: (0, i, 0)),
            pl.BlockSpec((2, RB), lambda i: (0, i)),
            pl.BlockSpec((D, H), lambda i: (0, 0)),
            pl.BlockSpec((1, H), lambda i: (0, 0)),
            pl.BlockSpec((H, OUT), lambda i: (0, 0)),
            pl.BlockSpec((1, OUT), lambda i: (0, 0)),
        ],
        out_specs=pl.BlockSpec((1, OUT), lambda i: (0, 0)),
        out_shape=jax.ShapeDtypeStruct((1, OUT), jnp.float32),
        scratch_shapes=[pltpu.VMEM((1, H), jnp.float32)],
        compiler_params=pltpu.CompilerParams(
            dimension_semantics=("arbitrary",)),
    )(agg_partials, norms, W1, b1_2d, Wfc, bfc_2d)


def kernel(x, edge_index, W1, b1, Wfc, bfc):
    src = edge_index[0]
    dst = edge_index[1]
    pad = jnp.full((EPAD - E,), N, dtype=jnp.int32)
    src_p = jnp.concatenate([src, pad])
    dst_p = jnp.concatenate([dst, pad])
    edg = jnp.stack([src_p.reshape(TOTAL_CHUNKS, C),
                     dst_p.reshape(TOTAL_CHUNKS, C)], axis=1)
    x_pad = jnp.zeros((NPAD, D), jnp.float32).at[:N].set(x)
    zeros_tile = jnp.zeros((ROWS_PER_SC_TILE, D), jnp.float32)
    b1_2d = b1.reshape(1, H)
    bfc_2d = bfc.reshape(1, OUT)

    degp = _sc_degrees(src, dst)
    xs, norms = _tc_prep(degp, x_pad)
    agg1 = _sc_agg(xs, edg, zeros_tile)
    h1s = _tc_layer(agg1, norms, W1, b1_2d)
    agg2 = _sc_agg(h1s, edg, zeros_tile)
    return _tc_final(agg2, norms, W1, b1_2d, Wfc, bfc_2d)
